# baseline (speedup 1.0000x reference)
import jax
import jax.numpy as jnp
from jax import lax
from jax.experimental import pallas as pl
from jax.experimental.pallas import tpu as pltpu

_BN_EPS = 1e-5
_VMEM_LIMIT = 64 * 1024 * 1024
_BF16 = jnp.bfloat16


def _stage1_kernel(x_ref, sc1_ref, sh1_ref, wm_ref, b0m_ref, wsc_ref,
                   y_ref, skip_ref, sum_ref, ssq_ref):
    f32 = jnp.float32
    x = x_ref[...].astype(f32)
    s, h, w, cin = x.shape
    cout4 = b0m_ref.shape[-1]
    cout = cout4 // 4

    sc1 = sc1_ref[...].astype(f32).reshape(s, 1, 1, cin)
    sh1 = sh1_ref[...].astype(f32).reshape(s, 1, 1, cin)
    a = jnp.maximum(x * sc1 + sh1, 0.0)
    ab = a.astype(_BF16)

    zrow = jnp.zeros((s, 1, w, cin), _BF16)
    zcol = jnp.zeros((s, h, 1, cin), _BF16)
    a_h = jnp.concatenate([ab[:, 1:], zrow], axis=1)
    a_w = jnp.concatenate([ab[:, :, 1:, :], zcol], axis=2)
    a_hw = jnp.concatenate([a_h[:, :, 1:, :], zcol], axis=2)

    lhs = jnp.concatenate([ab, a_w, a_h, a_hw], axis=-1
                          ).reshape(s * h * w, 4 * cin)
    p = jnp.dot(lhs, wm_ref[...], preferred_element_type=f32)
    p = p + b0m_ref[...].astype(f32)
    p3 = p.reshape(s, h * w, cout4)

    cs = jnp.sum(p3, axis=1, keepdims=True)
    qs = jnp.sum(p3 * p3, axis=1, keepdims=True)
    sum_ref[...] = (cs[..., 0:cout] + cs[..., cout:2 * cout] +
                    cs[..., 2 * cout:3 * cout] + cs[..., 3 * cout:]
                    ).astype(sum_ref.dtype)
    ssq_ref[...] = (qs[..., 0:cout] + qs[..., cout:2 * cout] +
                    qs[..., 2 * cout:3 * cout] + qs[..., 3 * cout:]
                    ).astype(ssq_ref.dtype)

    t = p.reshape(s, h, w, cout4)
    row0 = t[..., :2 * cout]
    row1 = t[..., 2 * cout:]
    y4 = jnp.stack([row0, row1], axis=2).reshape(s, 2 * h, w, 2 * cout)
    y_ref[...] = y4.astype(y_ref.dtype)

    xb = x.astype(_BF16)
    skip_ref[...] = jnp.dot(xb.reshape(s * h * w, cin), wsc_ref[...],
                            preferred_element_type=f32
                            ).reshape(s, h, w, cout).astype(skip_ref.dtype)


def _stage2_kernel(y_ref, sc2_ref, sh2_ref, w1g_ref, bias_ref, skip_ref,
                   o_ref):
    f32 = jnp.float32
    y = y_ref[...].astype(f32)
    s, ho, wo, c = y.shape
    cout = o_ref.shape[-1]

    sc2 = sc2_ref[...].astype(f32).reshape(s, 1, 1, c)
    sh2 = sh2_ref[...].astype(f32).reshape(s, 1, 1, c)
    a = jnp.maximum(y * sc2 + sh2, 0.0)
    ab = a.astype(_BF16)

    zrow = jnp.zeros((s, 1, wo, c), _BF16)
    zcol = jnp.zeros((s, ho + 2, 1, c), _BF16)
    ap = jnp.concatenate([zrow, ab, zrow], axis=1)
    ap = jnp.concatenate([zcol, ap, zcol], axis=2)

    sd = skip_ref[...]
    _, h, w, _ = sd.shape
    t = jnp.stack([sd, jnp.zeros_like(sd)], axis=3).reshape(s, h, 2 * w, cout)
    skip_up = jnp.stack([t, jnp.zeros_like(t)], axis=2
                        ).reshape(s * ho * wo, cout)

    rows = ap[:, 1:1 + ho]
    patch = jnp.concatenate(
        [rows[:, :, 0:wo, :], rows[:, :, 1:wo + 1, :],
         rows[:, :, 2:wo + 2, :]], axis=-1)
    patch_aug = jnp.concatenate(
        [patch.reshape(s * ho * wo, 3 * c), skip_up], axis=-1)
    q = jnp.dot(patch_aug, w1g_ref[...],
                preferred_element_type=f32)
    q = q.reshape(s, ho * wo, 3 * cout)
    q0 = q[..., 0:cout]
    q1 = q[..., cout:2 * cout]
    q2 = q[..., 2 * cout:]
    zr = jnp.zeros((s, wo, cout), f32)
    acc = (q1 + jnp.concatenate([zr, q0[:, :-wo]], axis=1)
           + jnp.concatenate([q2[:, wo:], zr], axis=1))
    out = acc.reshape(s, ho, wo, cout) + bias_ref[...].astype(f32)

    o_ref[...] = out.astype(o_ref.dtype)


def kernel(x, cond, wg1_t, wb1_t, wg2_t, wb2_t, w0, b0, w1, b1, wsc, bsc):
    f32 = jnp.float32
    xh = jnp.transpose(x, (0, 2, 3, 1))
    n, h, w, cin = xh.shape
    cout = b0.shape[0]
    hw = h * w
    sps = 2 if n % 2 == 0 else 1
    sps1 = 4 if n % 4 == 0 else sps

    xs = x.astype(f32)
    mean1 = jnp.mean(xs, axis=(0, 2, 3))
    var1 = jnp.mean(jnp.square(xs), axis=(0, 2, 3)) - jnp.square(mean1)
    inv1 = lax.rsqrt(var1 + _BN_EPS)
    gamma1 = cond.astype(f32) @ wg1_t
    beta1 = cond.astype(f32) @ wb1_t
    scale1 = (gamma1 * inv1).reshape(n, 1, cin)
    shift1 = (beta1 - gamma1 * mean1 * inv1).reshape(n, 1, cin)

    z = jnp.zeros((cin, cout), f32)
    row_a = jnp.concatenate([w0[1, 1], w0[1, 0], w0[0, 1], w0[0, 0]], axis=1)
    row_aw = jnp.concatenate([z, w0[1, 2], z, w0[0, 2]], axis=1)
    row_ah = jnp.concatenate([z, z, w0[2, 1], w0[2, 0]], axis=1)
    row_ahw = jnp.concatenate([z, z, z, w0[2, 2]], axis=1)
    wm = jnp.concatenate([row_a, row_aw, row_ah, row_ahw], axis=0).astype(_BF16)
    b0m = jnp.tile(b0, 4).reshape(1, 4 * cout)
    wsc_m = wsc[0, 0].astype(_BF16)

    y_fold, skip_half, s2, q2 = pl.pallas_call(
        _stage1_kernel,
        grid=(n // sps1,),
        in_specs=[
            pl.BlockSpec((sps1, h, w, cin), lambda i: (i, 0, 0, 0)),
            pl.BlockSpec((sps1, 1, cin), lambda i: (i, 0, 0)),
            pl.BlockSpec((sps1, 1, cin), lambda i: (i, 0, 0)),
            pl.BlockSpec((4 * cin, 4 * cout), lambda i: (0, 0)),
            pl.BlockSpec((1, 4 * cout), lambda i: (0, 0)),
            pl.BlockSpec((cin, cout), lambda i: (0, 0)),
        ],
        out_specs=(
            pl.BlockSpec((sps1, 2 * h, w, 2 * cout), lambda i: (i, 0, 0, 0)),
            pl.BlockSpec((sps1, h, w, cout), lambda i: (i, 0, 0, 0)),
            pl.BlockSpec((sps1, 1, cout), lambda i: (i, 0, 0)),
            pl.BlockSpec((sps1, 1, cout), lambda i: (i, 0, 0)),
        ),
        out_shape=(
            jax.ShapeDtypeStruct((n, 2 * h, w, 2 * cout), _BF16),
            jax.ShapeDtypeStruct((n, h, w, cout), _BF16),
            jax.ShapeDtypeStruct((n, 1, cout), f32),
            jax.ShapeDtypeStruct((n, 1, cout), f32),
        ),
        compiler_params=pltpu.CompilerParams(
            dimension_semantics=("parallel",),
            vmem_limit_bytes=_VMEM_LIMIT),
    )(xh, scale1, shift1, wm, b0m, wsc_m)

    y = y_fold.reshape(n, 2 * h, 2 * w, cout)

    count = jnp.asarray(n * (2 * h) * (2 * w), f32)
    mean2 = jnp.sum(s2, axis=(0, 1)) / count
    var2 = jnp.maximum(jnp.sum(q2, axis=(0, 1)) / count - jnp.square(mean2), 0.0)
    inv2 = lax.rsqrt(var2 + _BN_EPS)
    gamma2 = cond.astype(f32) @ wg2_t
    beta2 = cond.astype(f32) @ wb2_t
    scale2 = (gamma2 * inv2).reshape(n, 1, cout)
    shift2 = (beta2 - gamma2 * mean2 * inv2).reshape(n, 1, cout)

    w1r = w1.reshape(3, 3 * cout, cout)
    zc = jnp.zeros((cout, cout), f32)
    skip_rows = jnp.concatenate([zc, jnp.eye(cout, dtype=f32), zc], axis=1)
    w1g = jnp.concatenate(
        [jnp.concatenate([w1r[0], w1r[1], w1r[2]], axis=1), skip_rows],
        axis=0).astype(_BF16)
    bias_total = (b1 + bsc).reshape(1, cout)

    out = pl.pallas_call(
        _stage2_kernel,
        grid=(n // sps,),
        in_specs=[
            pl.BlockSpec((sps, 2 * h, 2 * w, cout), lambda i: (i, 0, 0, 0)),
            pl.BlockSpec((sps, 1, cout), lambda i: (i, 0, 0)),
            pl.BlockSpec((sps, 1, cout), lambda i: (i, 0, 0)),
            pl.BlockSpec((4 * cout, 3 * cout), lambda i: (0, 0)),
            pl.BlockSpec((1, cout), lambda i: (0, 0)),
            pl.BlockSpec((sps, h, w, cout), lambda i: (i, 0, 0, 0)),
        ],
        out_specs=pl.BlockSpec((sps, 2 * h, 2 * w, cout), lambda i: (i, 0, 0, 0)),
        out_shape=jax.ShapeDtypeStruct((n, 2 * h, 2 * w, cout), x.dtype),
        compiler_params=pltpu.CompilerParams(
            dimension_semantics=("parallel",),
            vmem_limit_bytes=_VMEM_LIMIT),
    )(y, scale2, shift2, w1g, bias_total, skip_half)

    return jnp.transpose(out, (0, 3, 1, 2))


# speedup vs baseline: 1.0097x; 1.0097x over previous
import jax
import jax.numpy as jnp
from jax import lax
from jax.experimental import pallas as pl
from jax.experimental.pallas import tpu as pltpu

_BN_EPS = 1e-5
_VMEM_LIMIT = 64 * 1024 * 1024
_BF16 = jnp.bfloat16


def _stage1_kernel(x_ref, sc1_ref, sh1_ref, wm_ref, b0m_ref, wsc_ref,
                   y_ref, skip_ref, sum_ref, ssq_ref):
    f32 = jnp.float32
    x = x_ref[...].astype(f32)
    s, h, w, cin = x.shape
    cout4 = b0m_ref.shape[-1]
    cout = cout4 // 4

    sc1 = sc1_ref[...].astype(f32).reshape(s, 1, 1, cin)
    sh1 = sh1_ref[...].astype(f32).reshape(s, 1, 1, cin)
    a = jnp.maximum(x * sc1 + sh1, 0.0)
    ab = a.astype(_BF16)

    zrow = jnp.zeros((s, 1, w, cin), _BF16)
    zcol = jnp.zeros((s, h, 1, cin), _BF16)
    a_h = jnp.concatenate([ab[:, 1:], zrow], axis=1)
    a_w = jnp.concatenate([ab[:, :, 1:, :], zcol], axis=2)
    a_hw = jnp.concatenate([a_h[:, :, 1:, :], zcol], axis=2)

    lhs = jnp.concatenate([ab, a_w, a_h, a_hw], axis=-1
                          ).reshape(s * h * w, 4 * cin)
    p = jnp.dot(lhs, wm_ref[...], preferred_element_type=f32)
    p = p + b0m_ref[...].astype(f32)
    p3 = p.reshape(s, h * w, cout4)

    cs = jnp.sum(p3, axis=1, keepdims=True)
    qs = jnp.sum(p3 * p3, axis=1, keepdims=True)
    sum_ref[...] = (cs[..., 0:cout] + cs[..., cout:2 * cout] +
                    cs[..., 2 * cout:3 * cout] + cs[..., 3 * cout:]
                    ).astype(sum_ref.dtype)
    ssq_ref[...] = (qs[..., 0:cout] + qs[..., cout:2 * cout] +
                    qs[..., 2 * cout:3 * cout] + qs[..., 3 * cout:]
                    ).astype(ssq_ref.dtype)

    t = p.reshape(s, h, w, cout4)
    row0 = t[..., :2 * cout]
    row1 = t[..., 2 * cout:]
    y4 = jnp.stack([row0, row1], axis=2).reshape(s, 2 * h, w, 2 * cout)
    y_ref[...] = y4.astype(y_ref.dtype)

    xb = x.astype(_BF16)
    skip_ref[...] = jnp.dot(xb.reshape(s * h * w, cin), wsc_ref[...],
                            preferred_element_type=f32
                            ).reshape(s, h, w, cout).astype(skip_ref.dtype)


def _stage2_kernel(y_ref, sc2_ref, sh2_ref, w1g_ref, bias_ref, skip_ref,
                   o_ref):
    f32 = jnp.float32
    y = y_ref[...].astype(f32)
    s, ho, wo, c = y.shape
    cout = o_ref.shape[-1]

    sc2 = sc2_ref[...].astype(f32).reshape(s, 1, 1, c)
    sh2 = sh2_ref[...].astype(f32).reshape(s, 1, 1, c)
    a = jnp.maximum(y * sc2 + sh2, 0.0)
    ab = a.astype(_BF16)

    zrow = jnp.zeros((s, 1, wo, c), _BF16)
    zcol = jnp.zeros((s, ho + 2, 1, c), _BF16)
    ap = jnp.concatenate([zrow, ab, zrow], axis=1)
    ap = jnp.concatenate([zcol, ap, zcol], axis=2)

    rows = ap[:, 1:1 + ho]
    patch = jnp.concatenate(
        [rows[:, :, 0:wo, :], rows[:, :, 1:wo + 1, :],
         rows[:, :, 2:wo + 2, :]], axis=-1)
    q = jnp.dot(patch.reshape(s * ho * wo, 3 * c), w1g_ref[...],
                preferred_element_type=f32)
    q = q.reshape(s, ho * wo, 3 * cout)
    q0 = q[..., 0:cout]
    q1 = q[..., cout:2 * cout]
    q2 = q[..., 2 * cout:]
    zr = jnp.zeros((s, wo, cout), f32)
    acc = (q1 + jnp.concatenate([zr, q0[:, :-wo]], axis=1)
           + jnp.concatenate([q2[:, wo:], zr], axis=1))
    out = acc.reshape(s, ho, wo, cout) + bias_ref[...].astype(f32)

    sd = skip_ref[...].astype(f32)
    _, h, w, _ = sd.shape
    t = jnp.stack([sd, jnp.zeros_like(sd)], axis=3).reshape(s, h, 2 * w, cout)
    skip_up = jnp.stack([t, jnp.zeros_like(t)], axis=2
                        ).reshape(s, 2 * h, 2 * w, cout)

    o_ref[...] = (out + skip_up).astype(o_ref.dtype)


def kernel(x, cond, wg1_t, wb1_t, wg2_t, wb2_t, w0, b0, w1, b1, wsc, bsc):
    f32 = jnp.float32
    xh = jnp.transpose(x, (0, 2, 3, 1))
    n, h, w, cin = xh.shape
    cout = b0.shape[0]
    hw = h * w
    sps = 2 if n % 2 == 0 else 1
    sps1 = 4 if n % 4 == 0 else sps

    xs = x.astype(f32)
    mean1 = jnp.mean(xs, axis=(0, 2, 3))
    var1 = jnp.mean(jnp.square(xs), axis=(0, 2, 3)) - jnp.square(mean1)
    inv1 = lax.rsqrt(var1 + _BN_EPS)
    gamma1 = cond.astype(f32) @ wg1_t
    beta1 = cond.astype(f32) @ wb1_t
    scale1 = (gamma1 * inv1).reshape(n, 1, cin)
    shift1 = (beta1 - gamma1 * mean1 * inv1).reshape(n, 1, cin)

    z = jnp.zeros((cin, cout), f32)
    row_a = jnp.concatenate([w0[1, 1], w0[1, 0], w0[0, 1], w0[0, 0]], axis=1)
    row_aw = jnp.concatenate([z, w0[1, 2], z, w0[0, 2]], axis=1)
    row_ah = jnp.concatenate([z, z, w0[2, 1], w0[2, 0]], axis=1)
    row_ahw = jnp.concatenate([z, z, z, w0[2, 2]], axis=1)
    wm = jnp.concatenate([row_a, row_aw, row_ah, row_ahw], axis=0).astype(_BF16)
    b0m = jnp.tile(b0, 4).reshape(1, 4 * cout)
    wsc_m = wsc[0, 0].astype(_BF16)

    y_fold, skip_half, s2, q2 = pl.pallas_call(
        _stage1_kernel,
        grid=(n // sps1,),
        in_specs=[
            pl.BlockSpec((sps1, h, w, cin), lambda i: (i, 0, 0, 0)),
            pl.BlockSpec((sps1, 1, cin), lambda i: (i, 0, 0)),
            pl.BlockSpec((sps1, 1, cin), lambda i: (i, 0, 0)),
            pl.BlockSpec((4 * cin, 4 * cout), lambda i: (0, 0)),
            pl.BlockSpec((1, 4 * cout), lambda i: (0, 0)),
            pl.BlockSpec((cin, cout), lambda i: (0, 0)),
        ],
        out_specs=(
            pl.BlockSpec((sps1, 2 * h, w, 2 * cout), lambda i: (i, 0, 0, 0)),
            pl.BlockSpec((sps1, h, w, cout), lambda i: (i, 0, 0, 0)),
            pl.BlockSpec((sps1, 1, cout), lambda i: (i, 0, 0)),
            pl.BlockSpec((sps1, 1, cout), lambda i: (i, 0, 0)),
        ),
        out_shape=(
            jax.ShapeDtypeStruct((n, 2 * h, w, 2 * cout), _BF16),
            jax.ShapeDtypeStruct((n, h, w, cout), _BF16),
            jax.ShapeDtypeStruct((n, 1, cout), f32),
            jax.ShapeDtypeStruct((n, 1, cout), f32),
        ),
        compiler_params=pltpu.CompilerParams(
            dimension_semantics=("parallel",),
            vmem_limit_bytes=_VMEM_LIMIT),
    )(xh, scale1, shift1, wm, b0m, wsc_m)

    y = y_fold.reshape(n, 2 * h, 2 * w, cout)

    count = jnp.asarray(n * (2 * h) * (2 * w), f32)
    mean2 = jnp.sum(s2, axis=(0, 1)) / count
    var2 = jnp.maximum(jnp.sum(q2, axis=(0, 1)) / count - jnp.square(mean2), 0.0)
    inv2 = lax.rsqrt(var2 + _BN_EPS)
    gamma2 = cond.astype(f32) @ wg2_t
    beta2 = cond.astype(f32) @ wb2_t
    scale2 = (gamma2 * inv2).reshape(n, 1, cout)
    shift2 = (beta2 - gamma2 * mean2 * inv2).reshape(n, 1, cout)

    w1r = w1.reshape(3, 3 * cout, cout)
    w1g = jnp.concatenate([w1r[0], w1r[1], w1r[2]], axis=1).astype(_BF16)
    bias_total = (b1 + bsc).reshape(1, cout)

    out = pl.pallas_call(
        _stage2_kernel,
        grid=(n // sps,),
        in_specs=[
            pl.BlockSpec((sps, 2 * h, 2 * w, cout), lambda i: (i, 0, 0, 0)),
            pl.BlockSpec((sps, 1, cout), lambda i: (i, 0, 0)),
            pl.BlockSpec((sps, 1, cout), lambda i: (i, 0, 0)),
            pl.BlockSpec((3 * cout, 3 * cout), lambda i: (0, 0)),
            pl.BlockSpec((1, cout), lambda i: (0, 0)),
            pl.BlockSpec((sps, h, w, cout), lambda i: (i, 0, 0, 0)),
        ],
        out_specs=pl.BlockSpec((sps, 2 * h, 2 * w, cout), lambda i: (i, 0, 0, 0)),
        out_shape=jax.ShapeDtypeStruct((n, 2 * h, 2 * w, cout), x.dtype),
        compiler_params=pltpu.CompilerParams(
            dimension_semantics=("parallel",),
            vmem_limit_bytes=_VMEM_LIMIT),
    )(y, scale2, shift2, w1g, bias_total, skip_half)

    return jnp.transpose(out, (0, 3, 1, 2))
